# baseline (speedup 1.0000x reference)
import functools

import jax
import jax.numpy as jnp
from jax import lax
from jax.experimental import pallas as pl
from jax.experimental.pallas import tpu as pltpu
from jax.experimental.pallas import tpu_sc as plsc

L = 16
NW = 32
CH = 2000
GROUPS = CH // L
N_BINS = 15


def _ece_body(nchunks, logits_hbm, labels_hbm, bt_hbm, out_hbm,
              lg_v, lb_v, bt_v, acc_v):
    cid = lax.axis_index("c")
    sid = lax.axis_index("s")
    wid = sid * 2 + cid

    zeros = jnp.zeros((L,), jnp.float32)
    ones = jnp.full((L,), 1.0, jnp.float32)
    for i in range(3 * N_BINS):
        acc_v[pl.ds(i * L, L)] = zeros

    pltpu.sync_copy(bt_hbm, bt_v)

    lane = lax.broadcasted_iota(jnp.int32, (L,), 0)
    lane3 = lane * 3

    def grp(g, _):
        b3 = g * (3 * L)
        i0 = lane3 + b3
        l0 = plsc.load_gather(lg_v, [i0])
        l1 = plsc.load_gather(lg_v, [i0 + 1])
        l2 = plsc.load_gather(lg_v, [i0 + 2])
        lb = lb_v[pl.ds(g * L, L)]

        m01 = jnp.maximum(l0, l1)
        lmax = jnp.maximum(m01, l2)
        s = jnp.exp(l0 - lmax) + jnp.exp(l1 - lmax) + jnp.exp(l2 - lmax)
        conf = 1.0 / s
        pred = jnp.where(l1 > l0, 1, 0).astype(jnp.int32)
        pred = jnp.where(l2 > m01, 2, pred)
        accf = jnp.where(pred == lb, 1.0, 0.0).astype(jnp.float32)

        b0 = jnp.minimum((conf * 15.0).astype(jnp.int32), N_BINS - 1)
        lo = plsc.load_gather(bt_v, [b0])
        hi = plsc.load_gather(bt_v, [b0 + 1])
        b = b0 - jnp.where(conf <= lo, 1, 0) + jnp.where(conf > hi, 1, 0)

        slot = b * L + lane
        plsc.addupdate_scatter(acc_v, [slot], ones)
        plsc.addupdate_scatter(acc_v, [slot + N_BINS * L], conf)
        plsc.addupdate_scatter(acc_v, [slot + 2 * N_BINS * L], accf)
        return 0

    def outer(j, _):
        c = wid + NW * j

        @pl.when(c < nchunks)
        def _():
            off3 = pl.multiple_of(c * (3 * CH), 8)
            off1 = pl.multiple_of(c * CH, 8)
            pltpu.sync_copy(logits_hbm.at[pl.ds(off3, 3 * CH)], lg_v)
            pltpu.sync_copy(labels_hbm.at[pl.ds(off1, CH)], lb_v)
            lax.fori_loop(0, GROUPS, grp, 0)

        return 0

    niter = (nchunks + NW - 1) // NW
    lax.fori_loop(0, niter, outer, 0)
    pltpu.sync_copy(acc_v, out_hbm.at[wid])


def kernel(logits, labels):
    n = logits.shape[0]
    assert n % CH == 0
    nchunks = n // CH

    bt = jnp.linspace(0.0, 1.0, N_BINS + 1).astype(jnp.float32)

    mesh = plsc.VectorSubcoreMesh(core_axis_name="c", subcore_axis_name="s", num_cores=2, num_subcores=16)
    run = pl.kernel(
        functools.partial(_ece_body, nchunks),
        out_type=jax.ShapeDtypeStruct((NW, 3 * N_BINS * L), jnp.float32),
        mesh=mesh,
        compiler_params=pltpu.CompilerParams(needs_layout_passes=False),
        scratch_types=[
            pltpu.VMEM((3 * CH,), jnp.float32),
            pltpu.VMEM((CH,), jnp.int32),
            pltpu.VMEM((L,), jnp.float32),
            pltpu.VMEM((3 * N_BINS * L,), jnp.float32),
        ],
    )
    parts = run(logits.reshape(-1), labels, bt)

    sums = parts.reshape(NW, 3, N_BINS, L).sum(axis=(0, 3))
    cnt, sconf, sacc = sums[0], sums[1], sums[2]
    n_total = jnp.asarray(n, dtype=jnp.float32)
    prop = cnt / n_total
    safe = jnp.maximum(cnt, 1.0)
    contrib = jnp.abs(sconf / safe - sacc / safe) * prop
    return jnp.sum(jnp.where(prop > 0.0, contrib, 0.0)).astype(jnp.float32)


# speedup vs baseline: 18.6569x; 18.6569x over previous
import functools

import jax
import jax.numpy as jnp
from jax import lax
from jax.experimental import pallas as pl
from jax.experimental.pallas import tpu as pltpu
from jax.experimental.pallas import tpu_sc as plsc

L = 16
NW = 32
CH = 2000
GROUPS = CH // L
N_BINS = 15


def _ece_body(nchunks, l0_hbm, l1_hbm, l2_hbm, labels_hbm, bt_hbm, out_hbm,
              l0_v, l1_v, l2_v, lb_v, bt_v, cnt_v, cf_v, ac_v):
    cid = lax.axis_index("c")
    sid = lax.axis_index("s")
    wid = sid * 2 + cid

    zeros = jnp.zeros((L,), jnp.float32)
    ones = jnp.full((L,), 1.0, jnp.float32)
    for i in range(N_BINS):
        cnt_v[pl.ds(i * L, L)] = zeros
        cf_v[pl.ds(i * L, L)] = zeros
        ac_v[pl.ds(i * L, L)] = zeros

    pltpu.sync_copy(bt_hbm, bt_v)

    lane = lax.broadcasted_iota(jnp.int32, (L,), 0)

    def grp(g, _):
        base = g * L
        l0 = l0_v[pl.ds(base, L)]
        l1 = l1_v[pl.ds(base, L)]
        l2 = l2_v[pl.ds(base, L)]
        lb = lb_v[pl.ds(base, L)]

        m01 = jnp.maximum(l0, l1)
        lmax = jnp.maximum(m01, l2)
        s = jnp.exp(l0 - lmax) + jnp.exp(l1 - lmax) + jnp.exp(l2 - lmax)
        conf = 1.0 / s
        pred = jnp.where(l1 > l0, 1, 0).astype(jnp.int32)
        pred = jnp.where(l2 > m01, 2, pred)
        accf = jnp.where(pred == lb, 1.0, 0.0).astype(jnp.float32)

        b0 = jnp.minimum((conf * 15.0).astype(jnp.int32), N_BINS - 1)
        lo = plsc.load_gather(bt_v, [b0])
        hi = plsc.load_gather(bt_v, [b0 + 1])
        b = b0 - jnp.where(conf <= lo, 1, 0) + jnp.where(conf > hi, 1, 0)

        slot = b * L + lane
        plsc.addupdate_scatter(cnt_v, [slot], ones)
        plsc.addupdate_scatter(cf_v, [slot], conf)
        plsc.addupdate_scatter(ac_v, [slot], accf)
        return 0

    def outer(j, _):
        c = wid + NW * j

        @pl.when(c < nchunks)
        def _():
            off = pl.multiple_of(c * CH, 8)
            pltpu.sync_copy(l0_hbm.at[pl.ds(off, CH)], l0_v)
            pltpu.sync_copy(l1_hbm.at[pl.ds(off, CH)], l1_v)
            pltpu.sync_copy(l2_hbm.at[pl.ds(off, CH)], l2_v)
            pltpu.sync_copy(labels_hbm.at[pl.ds(off, CH)], lb_v)
            lax.fori_loop(0, GROUPS, grp, 0)

        return 0

    niter = (nchunks + NW - 1) // NW
    lax.fori_loop(0, niter, outer, 0)
    pltpu.sync_copy(cnt_v, out_hbm.at[3 * wid])
    pltpu.sync_copy(cf_v, out_hbm.at[3 * wid + 1])
    pltpu.sync_copy(ac_v, out_hbm.at[3 * wid + 2])


def kernel(logits, labels):
    n = logits.shape[0]
    assert n % CH == 0
    nchunks = n // CH

    bt = jnp.linspace(0.0, 1.0, N_BINS + 1).astype(jnp.float32)
    l0 = logits[:, 0]
    l1 = logits[:, 1]
    l2 = logits[:, 2]

    mesh = plsc.VectorSubcoreMesh(
        core_axis_name="c", subcore_axis_name="s", num_cores=2, num_subcores=16
    )
    run = pl.kernel(
        functools.partial(_ece_body, nchunks),
        out_type=jax.ShapeDtypeStruct((NW * 3, N_BINS * L), jnp.float32),
        mesh=mesh,
        compiler_params=pltpu.CompilerParams(needs_layout_passes=False),
        scratch_types=[
            pltpu.VMEM((CH,), jnp.float32),
            pltpu.VMEM((CH,), jnp.float32),
            pltpu.VMEM((CH,), jnp.float32),
            pltpu.VMEM((CH,), jnp.int32),
            pltpu.VMEM((L,), jnp.float32),
            pltpu.VMEM((N_BINS * L,), jnp.float32),
            pltpu.VMEM((N_BINS * L,), jnp.float32),
            pltpu.VMEM((N_BINS * L,), jnp.float32),
        ],
    )
    parts = run(l0, l1, l2, labels, bt)

    sums = parts.reshape(NW, 3, N_BINS, L).sum(axis=(0, 3))
    cnt, sconf, sacc = sums[0], sums[1], sums[2]
    n_total = jnp.asarray(n, dtype=jnp.float32)
    prop = cnt / n_total
    safe = jnp.maximum(cnt, 1.0)
    contrib = jnp.abs(sconf / safe - sacc / safe) * prop
    return jnp.sum(jnp.where(prop > 0.0, contrib, 0.0)).astype(jnp.float32)


# speedup vs baseline: 44.8435x; 2.4036x over previous
import functools

import jax
import jax.numpy as jnp
from jax import lax
from jax.experimental import pallas as pl
from jax.experimental.pallas import tpu as pltpu
from jax.experimental.pallas import tpu_sc as plsc

L = 16
NW = 32
CH = 2000
GROUPS = CH // L
N_BINS = 15
C15 = 1.0 / 15.0


def _ece_body(nchunks, l0_hbm, l1_hbm, l2_hbm, lab_hbm, out_hbm,
              l0a, l1a, l2a, lba, l0b, l1b, l2b, lbb,
              cnt_v, cf_v, ac_v, sem0, sem1):
    cid = lax.axis_index("c")
    sid = lax.axis_index("s")
    wid = sid * 2 + cid

    zeros = jnp.zeros((L,), jnp.float32)
    ones = jnp.full((L,), 1.0, jnp.float32)
    for i in range(N_BINS):
        cnt_v[pl.ds(i * L, L)] = zeros
        cf_v[pl.ds(i * L, L)] = zeros
        ac_v[pl.ds(i * L, L)] = zeros

    lane = lax.broadcasted_iota(jnp.int32, (L,), 0)
    bufs = ((l0a, l1a, l2a, lba), (l0b, l1b, l2b, lbb))
    sems = (sem0, sem1)

    def _copies(j, b):
        c = wid + NW * j
        off = pl.multiple_of(c * CH, 8)
        srcs = (l0_hbm, l1_hbm, l2_hbm, lab_hbm)
        return c, [
            pltpu.make_async_copy(s.at[pl.ds(off, CH)], d, sems[b])
            for s, d in zip(srcs, bufs[b])
        ]

    def start(j, b):
        c, copies = _copies(j, b)

        @pl.when(c < nchunks)
        def _():
            for cp in copies:
                cp.start()

    def wait(j, b):
        c, copies = _copies(j, b)

        @pl.when(c < nchunks)
        def _():
            for cp in copies:
                cp.wait()

    def compute(j, b):
        c = wid + NW * j
        l0_v, l1_v, l2_v, lb_v = bufs[b]

        @pl.when(c < nchunks)
        def _():
            @plsc.parallel_loop(0, GROUPS, unroll=4)
            def grp(g):
                base = g * L
                l0 = l0_v[pl.ds(base, L)]
                l1 = l1_v[pl.ds(base, L)]
                l2 = l2_v[pl.ds(base, L)]
                lb = lb_v[pl.ds(base, L)]

                m01 = jnp.maximum(l0, l1)
                lmax = jnp.maximum(m01, l2)
                s = (jnp.exp(l0 - lmax) + jnp.exp(l1 - lmax)
                     + jnp.exp(l2 - lmax))
                conf = 1.0 / s
                pred = jnp.where(l1 > l0, 1, 0).astype(jnp.int32)
                pred = jnp.where(l2 > m01, 2, pred)
                accf = jnp.where(pred == lb, 1.0, 0.0).astype(jnp.float32)

                b0 = jnp.minimum((conf * 15.0).astype(jnp.int32), N_BINS - 1)
                b0f = b0.astype(jnp.float32)
                lo = b0f * C15
                hi = (b0f + 1.0) * C15
                bb = b0 - jnp.where(conf <= lo, 1, 0) \
                    + jnp.where(conf > hi, 1, 0)

                slot = bb * L + lane
                plsc.addupdate_scatter(cnt_v, [slot], ones)
                plsc.addupdate_scatter(cf_v, [slot], conf)
                plsc.addupdate_scatter(ac_v, [slot], accf)

    niter = (nchunks + NW - 1) // NW
    niter2 = (niter + 2) // 2

    start(0, 0)
    start(1, 1)

    def outer(j2, _):
        for b in (0, 1):
            j = 2 * j2 + b
            wait(j, b)
            compute(j, b)
            start(j + 2, b)
        return 0

    lax.fori_loop(0, niter2, outer, 0)
    pltpu.sync_copy(cnt_v, out_hbm.at[3 * wid])
    pltpu.sync_copy(cf_v, out_hbm.at[3 * wid + 1])
    pltpu.sync_copy(ac_v, out_hbm.at[3 * wid + 2])


def kernel(logits, labels):
    n = logits.shape[0]
    assert n % CH == 0
    nchunks = n // CH

    l0 = logits[:, 0]
    l1 = logits[:, 1]
    l2 = logits[:, 2]

    mesh = plsc.VectorSubcoreMesh(
        core_axis_name="c", subcore_axis_name="s", num_cores=2, num_subcores=16
    )
    run = pl.kernel(
        functools.partial(_ece_body, nchunks),
        out_type=jax.ShapeDtypeStruct((NW * 3, N_BINS * L), jnp.float32),
        mesh=mesh,
        compiler_params=pltpu.CompilerParams(needs_layout_passes=False),
        scratch_types=[
            pltpu.VMEM((CH,), jnp.float32),
            pltpu.VMEM((CH,), jnp.float32),
            pltpu.VMEM((CH,), jnp.float32),
            pltpu.VMEM((CH,), jnp.int32),
            pltpu.VMEM((CH,), jnp.float32),
            pltpu.VMEM((CH,), jnp.float32),
            pltpu.VMEM((CH,), jnp.float32),
            pltpu.VMEM((CH,), jnp.int32),
            pltpu.VMEM((N_BINS * L,), jnp.float32),
            pltpu.VMEM((N_BINS * L,), jnp.float32),
            pltpu.VMEM((N_BINS * L,), jnp.float32),
            pltpu.SemaphoreType.DMA,
            pltpu.SemaphoreType.DMA,
        ],
    )
    parts = run(l0, l1, l2, labels)

    sums = parts.reshape(NW, 3, N_BINS, L).sum(axis=(0, 3))
    cnt, sconf, sacc = sums[0], sums[1], sums[2]
    n_total = jnp.asarray(n, dtype=jnp.float32)
    prop = cnt / n_total
    safe = jnp.maximum(cnt, 1.0)
    contrib = jnp.abs(sconf / safe - sacc / safe) * prop
    return jnp.sum(jnp.where(prop > 0.0, contrib, 0.0)).astype(jnp.float32)
